# baseline (speedup 1.0000x reference)
import functools

import jax
import jax.numpy as jnp
from jax.experimental import pallas as pl

UNIQUE_TOKENS = 1000
CONTEXT = 2
EMB = 256
HID = 1024
B = 1024
T = 50
NC = T - 2 * CONTEXT
ROWS = B * NC

BLK = 368


def _body(codes_ref, table_ref, w1_ref, b1_ref, w2_ref, b2_ref, out_ref):
    codes = codes_ref[...]
    iota = jax.lax.broadcasted_iota(jnp.int32, (BLK, UNIQUE_TOKENS), 1)
    m = jnp.zeros((BLK, UNIQUE_TOKENS), jnp.bfloat16)
    for d in range(4):
        m = m + (codes[:, d : d + 1] == iota).astype(jnp.bfloat16)
    ctx = jax.lax.dot_general(
        m, table_ref[...], (((1,), (0,)), ((), ())),
        preferred_element_type=jnp.float32,
    )
    h = jax.lax.dot_general(
        ctx.astype(jnp.bfloat16), w1_ref[...], (((1,), (0,)), ((), ())),
        preferred_element_type=jnp.float32,
    )
    h = jnp.maximum(h + b1_ref[...], 0.0)
    out = jax.lax.dot_general(
        h.astype(jnp.bfloat16), w2_ref[...], (((1,), (0,)), ((), ())),
        preferred_element_type=jnp.float32,
    )
    out_ref[...] = out + b2_ref[...]


@jax.jit
def kernel(batchCode, table, W1, b1, W2, b2):
    c = CONTEXT
    offs = [d for d in range(2 * c + 1) if d != c]
    codes = jnp.stack([batchCode[:, d : d + NC] for d in offs], axis=-1)
    codes = codes.reshape(ROWS, 2 * c).astype(jnp.int32)

    grid = (ROWS // BLK,)
    out = pl.pallas_call(
        _body,
        grid=grid,
        in_specs=[
            pl.BlockSpec((BLK, 2 * c), lambda i: (i, 0)),
            pl.BlockSpec((UNIQUE_TOKENS, EMB), lambda i: (0, 0)),
            pl.BlockSpec((EMB, HID), lambda i: (0, 0)),
            pl.BlockSpec((1, HID), lambda i: (0, 0)),
            pl.BlockSpec((HID, UNIQUE_TOKENS), lambda i: (0, 0)),
            pl.BlockSpec((1, UNIQUE_TOKENS), lambda i: (0, 0)),
        ],
        out_specs=pl.BlockSpec((BLK, UNIQUE_TOKENS), lambda i: (i, 0)),
        out_shape=jax.ShapeDtypeStruct((ROWS, UNIQUE_TOKENS), jnp.float32),
    )(
        codes,
        table.astype(jnp.bfloat16),
        W1.astype(jnp.bfloat16),
        b1.reshape(1, HID),
        W2.astype(jnp.bfloat16),
        b2.reshape(1, UNIQUE_TOKENS),
    )
    return out.reshape(B, NC, UNIQUE_TOKENS)


# speedup vs baseline: 1.2778x; 1.2778x over previous
import functools

import jax
import jax.numpy as jnp
from jax.experimental import pallas as pl

UNIQUE_TOKENS = 1000
CONTEXT = 2
EMB = 256
HID = 1024
B = 1024
T = 50
NC = T - 2 * CONTEXT
ROWS = B * NC

BLK = 368


BB = BLK // NC


def _body(codes_ref, table_ref, w1_ref, b1_ref, w2_ref, b2_ref, out_ref):
    codes = codes_ref[...]
    iota = jax.lax.broadcasted_iota(jnp.int32, (BLK, UNIQUE_TOKENS), 1)
    m = jnp.zeros((BLK, UNIQUE_TOKENS), jnp.bfloat16)
    for d in range(4):
        m = m + (codes[:, d : d + 1] == iota).astype(jnp.bfloat16)
    ctx = jax.lax.dot_general(
        m, table_ref[...], (((1,), (0,)), ((), ())),
        preferred_element_type=jnp.float32,
    )
    h = jax.lax.dot_general(
        ctx.astype(jnp.bfloat16), w1_ref[...], (((1,), (0,)), ((), ())),
        preferred_element_type=jnp.float32,
    )
    h = jnp.maximum(h + b1_ref[...], 0.0)
    out = jax.lax.dot_general(
        h.astype(jnp.bfloat16), w2_ref[...], (((1,), (0,)), ((), ())),
        preferred_element_type=jnp.float32,
    )
    out = out + b2_ref[...]
    for b in range(BB):
        out_ref[b] = out[b * NC : (b + 1) * NC]


@jax.jit
def kernel(batchCode, table, W1, b1, W2, b2):
    c = CONTEXT
    offs = [d for d in range(2 * c + 1) if d != c]
    codes = jnp.stack([batchCode[:, d : d + NC] for d in offs], axis=-1)
    codes = codes.reshape(ROWS, 2 * c).astype(jnp.int32)

    grid = (ROWS // BLK,)
    out = pl.pallas_call(
        _body,
        grid=grid,
        in_specs=[
            pl.BlockSpec((BLK, 2 * c), lambda i: (i, 0)),
            pl.BlockSpec((UNIQUE_TOKENS, EMB), lambda i: (0, 0)),
            pl.BlockSpec((EMB, HID), lambda i: (0, 0)),
            pl.BlockSpec((1, HID), lambda i: (0, 0)),
            pl.BlockSpec((HID, UNIQUE_TOKENS), lambda i: (0, 0)),
            pl.BlockSpec((1, UNIQUE_TOKENS), lambda i: (0, 0)),
        ],
        out_specs=pl.BlockSpec((BB, NC, UNIQUE_TOKENS), lambda i: (i, 0, 0)),
        out_shape=jax.ShapeDtypeStruct((B, NC, UNIQUE_TOKENS), jnp.float32),
    )(
        codes,
        table.astype(jnp.bfloat16),
        W1.astype(jnp.bfloat16),
        b1.reshape(1, HID),
        W2.astype(jnp.bfloat16),
        b2.reshape(1, UNIQUE_TOKENS),
    )
    return out


# speedup vs baseline: 1.3958x; 1.0923x over previous
import functools

import jax
import jax.numpy as jnp
from jax import lax
from jax.experimental import pallas as pl
from jax.experimental.pallas import tpu as pltpu
from jax.experimental.pallas import tpu_sc as plsc

UNIQUE_TOKENS = 1000
CONTEXT = 2
EMB = 256
HID = 1024
B = 1024
T = 50
NC = T - 2 * CONTEXT
ROWS = B * NC
NROWS = B * T

BB = 8
BLK = BB * NC

_SC_INFO = plsc.get_sparse_core_info()
NCORE = _SC_INFO.num_cores
NSUB = _SC_INFO.num_subcores
NW = NCORE * NSUB
RPW = NROWS // NW
CHUNK = 80
NCH = RPW // CHUNK


def _gather_body(idx_hbm, table_hbm, out_hbm, idx_v, buf0, buf1, sem0, sem1):
    wid = lax.axis_index("s") * NCORE + lax.axis_index("c")
    pltpu.sync_copy(idx_hbm.at[wid], idx_v)
    bufs = (buf0, buf1)
    sems = (sem0, sem1)
    handles = [None] * NCH
    handles[0] = pltpu.async_copy(table_hbm.at[idx_v.at[0]], buf0, sem0)
    for c in range(NCH):
        if c + 1 < NCH:
            handles[c + 1] = pltpu.async_copy(
                table_hbm.at[idx_v.at[c + 1]], bufs[(c + 1) % 2], sems[(c + 1) % 2]
            )
        handles[c].wait()
        row0 = pl.multiple_of(wid * RPW + c * CHUNK, 8)
        pltpu.sync_copy(bufs[c % 2], out_hbm.at[pl.ds(row0, CHUNK)])


_gather = functools.partial(
    pl.kernel,
    mesh=plsc.VectorSubcoreMesh(core_axis_name="c", subcore_axis_name="s"),
    out_type=jax.ShapeDtypeStruct((NROWS, EMB), jnp.float32),
    scratch_types=[
        pltpu.VMEM((NCH, CHUNK), jnp.int32),

        pltpu.VMEM((CHUNK, EMB), jnp.float32),
        pltpu.VMEM((CHUNK, EMB), jnp.float32),
        pltpu.SemaphoreType.DMA,
        pltpu.SemaphoreType.DMA,
    ],
)(_gather_body)


def _mlp_body(e_ref, w1_ref, b1_ref, w2_ref, b2_ref, out_ref):
    e = e_ref[...]
    parts = []
    for b in range(BB):
        eb = e[b * T : (b + 1) * T]
        parts.append(eb[0:NC] + eb[1 : NC + 1] + eb[3 : NC + 3] + eb[4 : NC + 4])
    ctx = jnp.concatenate(parts, axis=0)
    h = jax.lax.dot_general(
        ctx.astype(jnp.bfloat16), w1_ref[...], (((1,), (0,)), ((), ())),
        preferred_element_type=jnp.float32,
    )
    h = jnp.maximum(h + b1_ref[...], 0.0)
    out = jax.lax.dot_general(
        h.astype(jnp.bfloat16), w2_ref[...], (((1,), (0,)), ((), ())),
        preferred_element_type=jnp.float32,
    )
    out = out + b2_ref[...]
    for b in range(BB):
        out_ref[b] = out[b * NC : (b + 1) * NC]


@jax.jit
def kernel(batchCode, table, W1, b1, W2, b2):
    idx = batchCode.astype(jnp.int32).reshape(NW, NCH, CHUNK)
    e = _gather(idx, table)

    grid = (B // BB,)
    out = pl.pallas_call(
        _mlp_body,
        grid=grid,
        in_specs=[
            pl.BlockSpec((BB * T, EMB), lambda i: (i, 0)),
            pl.BlockSpec((EMB, HID), lambda i: (0, 0)),
            pl.BlockSpec((1, HID), lambda i: (0, 0)),
            pl.BlockSpec((HID, UNIQUE_TOKENS), lambda i: (0, 0)),
            pl.BlockSpec((1, UNIQUE_TOKENS), lambda i: (0, 0)),
        ],
        out_specs=pl.BlockSpec((BB, NC, UNIQUE_TOKENS), lambda i: (i, 0, 0)),
        out_shape=jax.ShapeDtypeStruct((B, NC, UNIQUE_TOKENS), jnp.float32),
    )(
        e,
        W1.astype(jnp.bfloat16),
        b1.reshape(1, HID),
        W2.astype(jnp.bfloat16),
        b2.reshape(1, UNIQUE_TOKENS),
    )
    return out


# speedup vs baseline: 2.5161x; 1.8026x over previous
import functools

import jax
import jax.numpy as jnp
from jax import lax
from jax.experimental import pallas as pl
from jax.experimental.pallas import tpu as pltpu
from jax.experimental.pallas import tpu_sc as plsc

UNIQUE_TOKENS = 1000
CONTEXT = 2
EMB = 256
HID = 1024
B = 1024
T = 50
NC = T - 2 * CONTEXT
NROWS = B * T

_SC_INFO = plsc.get_sparse_core_info()
NCORE = _SC_INFO.num_cores
NSUB = _SC_INFO.num_subcores
NW = NCORE * NSUB
RPW = NROWS // NW
CHUNK = 80
NCH = RPW // CHUNK


def _gather_body(idx_hbm, table_hbm, out_hbm, idx_v, buf0, buf1, sem0, sem1):
    wid = lax.axis_index("s") * NCORE + lax.axis_index("c")
    pltpu.sync_copy(idx_hbm.at[wid], idx_v)
    bufs = (buf0, buf1)
    sems = (sem0, sem1)
    handles = [None] * NCH
    handles[0] = pltpu.async_copy(table_hbm.at[idx_v.at[0]], buf0, sem0)
    for c in range(NCH):
        if c + 1 < NCH:
            handles[c + 1] = pltpu.async_copy(
                table_hbm.at[idx_v.at[c + 1]], bufs[(c + 1) % 2], sems[(c + 1) % 2]
            )
        handles[c].wait()
        row0 = pl.multiple_of(wid * RPW + c * CHUNK, 8)
        pltpu.sync_copy(bufs[c % 2], out_hbm.at[pl.ds(row0, CHUNK)])


_gather = functools.partial(
    pl.kernel,
    mesh=plsc.VectorSubcoreMesh(core_axis_name="c", subcore_axis_name="s"),
    out_type=jax.ShapeDtypeStruct((NROWS, EMB), jnp.float32),
    scratch_types=[
        pltpu.VMEM((NCH, CHUNK), jnp.int32),
        pltpu.VMEM((CHUNK, EMB), jnp.float32),
        pltpu.VMEM((CHUNK, EMB), jnp.float32),
        pltpu.SemaphoreType.DMA,
        pltpu.SemaphoreType.DMA,
    ],
)(_gather_body)


def _mlp_body(e0_ref, e1_ref, e3_ref, e4_ref, w1_ref, b1_ref, w2_ref, b2_ref, out_ref):
    ctx = e0_ref[0] + e1_ref[0] + e3_ref[0] + e4_ref[0]
    h = jax.lax.dot_general(
        w1_ref[...], ctx.astype(jnp.bfloat16), (((0,), (1,)), ((), ())),
        preferred_element_type=jnp.float32,
    )
    h = jnp.maximum(h + b1_ref[...], 0.0)
    out = jax.lax.dot_general(
        w2_ref[...], h.astype(jnp.bfloat16), (((0,), (0,)), ((), ())),
        preferred_element_type=jnp.float32,
    )
    out_ref[0] = out + b2_ref[...]


@jax.jit
def kernel(batchCode, table, W1, b1, W2, b2):
    idx = batchCode.astype(jnp.int32).T.reshape(NW, NCH, CHUNK)
    e = _gather(idx, table)
    e3 = e.reshape(T, B, EMB)

    grid = (NC,)
    slab = pl.BlockSpec((1, B, EMB), lambda i: (i, 0, 0))
    out_t = pl.pallas_call(
        _mlp_body,
        grid=grid,
        in_specs=[
            pl.BlockSpec((1, B, EMB), lambda i: (i, 0, 0)),
            pl.BlockSpec((1, B, EMB), lambda i: (i + 1, 0, 0)),
            pl.BlockSpec((1, B, EMB), lambda i: (i + 3, 0, 0)),
            pl.BlockSpec((1, B, EMB), lambda i: (i + 4, 0, 0)),
            pl.BlockSpec((EMB, HID), lambda i: (0, 0)),
            pl.BlockSpec((HID, 1), lambda i: (0, 0)),
            pl.BlockSpec((HID, UNIQUE_TOKENS), lambda i: (0, 0)),
            pl.BlockSpec((UNIQUE_TOKENS, 1), lambda i: (0, 0)),
        ],
        out_specs=pl.BlockSpec((1, UNIQUE_TOKENS, B), lambda i: (i, 0, 0)),
        out_shape=jax.ShapeDtypeStruct((NC, UNIQUE_TOKENS, B), jnp.float32),
    )(
        e3,
        e3,
        e3,
        e3,
        W1.astype(jnp.bfloat16),
        b1.reshape(HID, 1),
        W2.astype(jnp.bfloat16),
        b2.reshape(UNIQUE_TOKENS, 1),
    )
    return jnp.transpose(out_t, (2, 0, 1))


# speedup vs baseline: 2.5958x; 1.0317x over previous
import functools

import jax
import jax.numpy as jnp
from jax import lax
from jax.experimental import pallas as pl
from jax.experimental.pallas import tpu as pltpu
from jax.experimental.pallas import tpu_sc as plsc

UNIQUE_TOKENS = 1000
CONTEXT = 2
EMB = 256
HID = 1024
B = 1024
T = 50
NC = T - 2 * CONTEXT
NROWS = B * T
NSLOT = 6

_SC_INFO = plsc.get_sparse_core_info()
NCORE = _SC_INFO.num_cores
NSUB = _SC_INFO.num_subcores
NW = NCORE * NSUB
RPW = NROWS // NW
CHUNK = 80
NCH = RPW // CHUNK


def _gather_body(idx_hbm, table_hbm, out_hbm, idx_v, buf0, buf1, sem0, sem1):
    wid = lax.axis_index("s") * NCORE + lax.axis_index("c")
    pltpu.sync_copy(idx_hbm.at[wid], idx_v)
    bufs = (buf0, buf1)
    sems = (sem0, sem1)
    handles = [None] * NCH
    handles[0] = pltpu.async_copy(table_hbm.at[idx_v.at[0]], buf0, sem0)
    for c in range(NCH):
        if c + 1 < NCH:
            handles[c + 1] = pltpu.async_copy(
                table_hbm.at[idx_v.at[c + 1]], bufs[(c + 1) % 2], sems[(c + 1) % 2]
            )
        handles[c].wait()
        row0 = pl.multiple_of(wid * RPW + c * CHUNK, 8)
        pltpu.sync_copy(bufs[c % 2], out_hbm.at[pl.ds(row0, CHUNK)])


_gather = functools.partial(
    pl.kernel,
    mesh=plsc.VectorSubcoreMesh(core_axis_name="c", subcore_axis_name="s"),
    out_type=jax.ShapeDtypeStruct((NROWS, EMB), jnp.float32),
    scratch_types=[
        pltpu.VMEM((NCH, CHUNK), jnp.int32),
        pltpu.VMEM((CHUNK, EMB), jnp.float32),
        pltpu.VMEM((CHUNK, EMB), jnp.float32),
        pltpu.SemaphoreType.DMA,
        pltpu.SemaphoreType.DMA,
    ],
)(_gather_body)


def _slab_copy(e_hbm, ring, sems, t):
    return pltpu.make_async_copy(e_hbm.at[t], ring.at[t % NSLOT], sems.at[t % NSLOT])


def _mlp_body(e_hbm, w1_ref, b1_ref, w2_ref, b2_ref, out_ref, ring, sems):
    i = pl.program_id(0)

    @pl.when(i == 0)
    def _prime():
        for t in range(5):
            _slab_copy(e_hbm, ring, sems, t).start()
        for t in range(5):
            _slab_copy(e_hbm, ring, sems, t).wait()

    @pl.when(i + 5 < T)
    def _prefetch():
        _slab_copy(e_hbm, ring, sems, i + 5).start()

    @pl.when(i > 0)
    def _await_slab():
        _slab_copy(e_hbm, ring, sems, i + 4).wait()

    ctx = ring[i % NSLOT] + ring[(i + 1) % NSLOT] + ring[(i + 3) % NSLOT] + ring[(i + 4) % NSLOT]
    h = jax.lax.dot_general(
        w1_ref[...], ctx.astype(jnp.bfloat16), (((0,), (1,)), ((), ())),
        preferred_element_type=jnp.float32,
    )
    h = jnp.maximum(h + b1_ref[...], 0.0)
    out = jax.lax.dot_general(
        w2_ref[...], h.astype(jnp.bfloat16), (((0,), (0,)), ((), ())),
        preferred_element_type=jnp.float32,
    )
    out_ref[0] = out + b2_ref[...]


@jax.jit
def kernel(batchCode, table, W1, b1, W2, b2):
    idx = batchCode.astype(jnp.int32).T.reshape(NW, NCH, CHUNK)
    e = _gather(idx, table)
    e3 = e.reshape(T, B, EMB)

    grid = (NC,)
    out_t = pl.pallas_call(
        _mlp_body,
        grid=grid,
        in_specs=[
            pl.BlockSpec(memory_space=pl.ANY),
            pl.BlockSpec((EMB, HID), lambda i: (0, 0)),
            pl.BlockSpec((HID, 1), lambda i: (0, 0)),
            pl.BlockSpec((HID, UNIQUE_TOKENS), lambda i: (0, 0)),
            pl.BlockSpec((UNIQUE_TOKENS, 1), lambda i: (0, 0)),
        ],
        out_specs=pl.BlockSpec((1, UNIQUE_TOKENS, B), lambda i: (i, 0, 0)),
        out_shape=jax.ShapeDtypeStruct((NC, UNIQUE_TOKENS, B), jnp.float32),
        scratch_shapes=[
            pltpu.VMEM((NSLOT, B, EMB), jnp.float32),
            pltpu.SemaphoreType.DMA((NSLOT,)),
        ],
    )(
        e3,
        W1.astype(jnp.bfloat16),
        b1.reshape(HID, 1),
        W2.astype(jnp.bfloat16),
        b2.reshape(UNIQUE_TOKENS, 1),
    )
    return jnp.transpose(out_t, (2, 0, 1))


# speedup vs baseline: 2.6510x; 1.0213x over previous
import functools

import jax
import jax.numpy as jnp
from jax import lax
from jax.experimental import pallas as pl
from jax.experimental.pallas import tpu as pltpu
from jax.experimental.pallas import tpu_sc as plsc

UNIQUE_TOKENS = 1000
CONTEXT = 2
EMB = 256
HID = 1024
B = 1024
T = 50
NC = T - 2 * CONTEXT
NROWS = B * T
NSLOT = 6

_SC_INFO = plsc.get_sparse_core_info()
NCORE = _SC_INFO.num_cores
NSUB = _SC_INFO.num_subcores
NW = NCORE * NSUB
RPW = NROWS // NW
CHUNK = 80
NCH = RPW // CHUNK


def _gather_body(idx_hbm, table_hbm, out_hbm, idx_v, buf0, buf1, sem0, sem1):
    wid = lax.axis_index("s") * NCORE + lax.axis_index("c")
    pltpu.sync_copy(idx_hbm.at[wid], idx_v)
    bufs = (buf0, buf1)
    sems = (sem0, sem1)
    handles = [None] * NCH
    handles[0] = pltpu.async_copy(table_hbm.at[idx_v.at[0]], buf0, sem0)
    for c in range(NCH):
        if c + 1 < NCH:
            handles[c + 1] = pltpu.async_copy(
                table_hbm.at[idx_v.at[c + 1]], bufs[(c + 1) % 2], sems[(c + 1) % 2]
            )
        handles[c].wait()
        row0 = pl.multiple_of(wid * RPW + c * CHUNK, 8)
        pltpu.sync_copy(bufs[c % 2], out_hbm.at[pl.ds(row0, CHUNK)])


_gather = functools.partial(
    pl.kernel,
    mesh=plsc.VectorSubcoreMesh(core_axis_name="c", subcore_axis_name="s"),
    out_type=jax.ShapeDtypeStruct((NROWS, EMB), jnp.float32),
    scratch_types=[
        pltpu.VMEM((NCH, CHUNK), jnp.int32),
        pltpu.VMEM((CHUNK, EMB), jnp.float32),
        pltpu.VMEM((CHUNK, EMB), jnp.float32),
        pltpu.SemaphoreType.DMA,
        pltpu.SemaphoreType.DMA,
    ],
)(_gather_body)


def _slab_copy(e_hbm, ring, sems, t):
    return pltpu.make_async_copy(e_hbm.at[t], ring.at[t % NSLOT], sems.at[t % NSLOT])


def _mlp_body(e_hbm, w1_ref, b1_ref, w2_ref, b2_ref, out_ref, ring, sems):
    i = pl.program_id(0)

    @pl.when(i == 0)
    def _prime():
        for t in range(5):
            _slab_copy(e_hbm, ring, sems, t).start()
        for t in range(5):
            _slab_copy(e_hbm, ring, sems, t).wait()

    @pl.when(i + 5 < T)
    def _prefetch():
        _slab_copy(e_hbm, ring, sems, i + 5).start()

    @pl.when(i > 0)
    def _await_slab():
        _slab_copy(e_hbm, ring, sems, i + 4).wait()

    ctx = ring[i % NSLOT] + ring[(i + 1) % NSLOT] + ring[(i + 3) % NSLOT] + ring[(i + 4) % NSLOT]
    h = jax.lax.dot_general(
        w1_ref[...], ctx.astype(jnp.bfloat16), (((1,), (1,)), ((), ())),
        preferred_element_type=jnp.float32,
    )
    h = jnp.maximum(h + b1_ref[...], 0.0)
    out = jax.lax.dot_general(
        w2_ref[...], h.astype(jnp.bfloat16), (((1,), (0,)), ((), ())),
        preferred_element_type=jnp.float32,
    )
    out_ref[0] = out + b2_ref[...]


@jax.jit
def kernel(batchCode, table, W1, b1, W2, b2):
    idx = batchCode.astype(jnp.int32).T.reshape(NW, NCH, CHUNK)
    e = _gather(idx, table)
    e3 = e.reshape(T, B, EMB)

    grid = (NC,)
    out_t = pl.pallas_call(
        _mlp_body,
        grid=grid,
        in_specs=[
            pl.BlockSpec(memory_space=pl.ANY),
            pl.BlockSpec((HID, EMB), lambda i: (0, 0)),
            pl.BlockSpec((HID, 1), lambda i: (0, 0)),
            pl.BlockSpec((UNIQUE_TOKENS, HID), lambda i: (0, 0)),
            pl.BlockSpec((UNIQUE_TOKENS, 1), lambda i: (0, 0)),
        ],
        out_specs=pl.BlockSpec((1, UNIQUE_TOKENS, B), lambda i: (i, 0, 0)),
        out_shape=jax.ShapeDtypeStruct((NC, UNIQUE_TOKENS, B), jnp.float32),
        scratch_shapes=[
            pltpu.VMEM((NSLOT, B, EMB), jnp.float32),
            pltpu.SemaphoreType.DMA((NSLOT,)),
        ],
    )(
        e3,
        W1.T.astype(jnp.bfloat16),
        b1.reshape(HID, 1),
        W2.T.astype(jnp.bfloat16),
        b2.reshape(UNIQUE_TOKENS, 1),
    )
    return jnp.transpose(out_t, (2, 0, 1))


# speedup vs baseline: 2.7636x; 1.0425x over previous
import functools

import jax
import jax.numpy as jnp
from jax import lax
from jax.experimental import pallas as pl
from jax.experimental.pallas import tpu as pltpu
from jax.experimental.pallas import tpu_sc as plsc

UNIQUE_TOKENS = 1000
CONTEXT = 2
EMB = 256
HID = 1024
B = 1024
T = 50
NC = T - 2 * CONTEXT
NROWS = B * T
NSLOT = 8

_SC_INFO = plsc.get_sparse_core_info()
NCORE = _SC_INFO.num_cores
NSUB = _SC_INFO.num_subcores
NW = NCORE * NSUB
RPW = NROWS // NW
CHUNK = 80
NCH = RPW // CHUNK


def _gather_body(idx_hbm, table_hbm, out_hbm, idx_v, buf0, buf1, sem0, sem1):
    wid = lax.axis_index("s") * NCORE + lax.axis_index("c")
    pltpu.sync_copy(idx_hbm.at[wid], idx_v)
    bufs = (buf0, buf1)
    sems = (sem0, sem1)
    handles = [None] * NCH
    handles[0] = pltpu.async_copy(table_hbm.at[idx_v.at[0]], buf0, sem0)
    for c in range(NCH):
        if c + 1 < NCH:
            handles[c + 1] = pltpu.async_copy(
                table_hbm.at[idx_v.at[c + 1]], bufs[(c + 1) % 2], sems[(c + 1) % 2]
            )
        handles[c].wait()
        row0 = pl.multiple_of(wid * RPW + c * CHUNK, 8)
        pltpu.sync_copy(bufs[c % 2], out_hbm.at[pl.ds(row0, CHUNK)])


_gather = functools.partial(
    pl.kernel,
    mesh=plsc.VectorSubcoreMesh(core_axis_name="c", subcore_axis_name="s"),
    out_type=jax.ShapeDtypeStruct((NROWS, EMB), jnp.float32),
    scratch_types=[
        pltpu.VMEM((NCH, CHUNK), jnp.int32),
        pltpu.VMEM((CHUNK, EMB), jnp.float32),
        pltpu.VMEM((CHUNK, EMB), jnp.float32),
        pltpu.SemaphoreType.DMA,
        pltpu.SemaphoreType.DMA,
    ],
)(_gather_body)


def _slab_copy(e_hbm, ring, sems, t):
    return pltpu.make_async_copy(e_hbm.at[t], ring.at[t % NSLOT], sems.at[t % NSLOT])


def _mlp_body(e_hbm, w1_ref, b1_ref, w2_ref, b2_ref, out_ref, ring, sems):
    j = pl.program_id(0)
    t0 = 2 * j

    @pl.when(j == 0)
    def _prime():
        for t in range(6):
            _slab_copy(e_hbm, ring, sems, t).start()
        for t in range(6):
            _slab_copy(e_hbm, ring, sems, t).wait()

    @pl.when(t0 + 7 < T)
    def _prefetch():
        _slab_copy(e_hbm, ring, sems, t0 + 6).start()
        _slab_copy(e_hbm, ring, sems, t0 + 7).start()

    @pl.when(j > 0)
    def _await_slab():
        _slab_copy(e_hbm, ring, sems, t0 + 4).wait()
        _slab_copy(e_hbm, ring, sems, t0 + 5).wait()

    def mlp(ctx):
        h = jax.lax.dot_general(
            w1_ref[...], ctx.astype(jnp.bfloat16), (((1,), (1,)), ((), ())),
            preferred_element_type=jnp.float32,
        )
        h = jnp.maximum(h + b1_ref[...], 0.0)
        out = jax.lax.dot_general(
            w2_ref[...], h.astype(jnp.bfloat16), (((1,), (0,)), ((), ())),
            preferred_element_type=jnp.float32,
        )
        return out + b2_ref[...]

    shared = ring[(t0 + 1) % NSLOT] + ring[(t0 + 4) % NSLOT]
    out_ref[0] = mlp(shared + ring[t0 % NSLOT] + ring[(t0 + 3) % NSLOT])
    out_ref[1] = mlp(shared + ring[(t0 + 2) % NSLOT] + ring[(t0 + 5) % NSLOT])


@jax.jit
def kernel(batchCode, table, W1, b1, W2, b2):
    idx = batchCode.astype(jnp.int32).T.reshape(NW, NCH, CHUNK)
    e = _gather(idx, table)
    e3 = e.reshape(T, B, EMB)

    grid = (NC // 2,)
    out_t = pl.pallas_call(
        _mlp_body,
        grid=grid,
        in_specs=[
            pl.BlockSpec(memory_space=pl.ANY),
            pl.BlockSpec((HID, EMB), lambda i: (0, 0)),
            pl.BlockSpec((HID, 1), lambda i: (0, 0)),
            pl.BlockSpec((UNIQUE_TOKENS, HID), lambda i: (0, 0)),
            pl.BlockSpec((UNIQUE_TOKENS, 1), lambda i: (0, 0)),
        ],
        out_specs=pl.BlockSpec((2, UNIQUE_TOKENS, B), lambda i: (i, 0, 0)),
        out_shape=jax.ShapeDtypeStruct((NC, UNIQUE_TOKENS, B), jnp.float32),
        scratch_shapes=[
            pltpu.VMEM((NSLOT, B, EMB), jnp.float32),
            pltpu.SemaphoreType.DMA((NSLOT,)),
        ],
    )(
        e3,
        W1.T.astype(jnp.bfloat16),
        b1.reshape(HID, 1),
        W2.T.astype(jnp.bfloat16),
        b2.reshape(UNIQUE_TOKENS, 1),
    )
    return jnp.transpose(out_t, (2, 0, 1))


# speedup vs baseline: 2.8704x; 1.0387x over previous
import functools

import jax
import jax.numpy as jnp
from jax import lax
from jax.experimental import pallas as pl
from jax.experimental.pallas import tpu as pltpu
from jax.experimental.pallas import tpu_sc as plsc

UNIQUE_TOKENS = 1000
CONTEXT = 2
EMB = 256
HID = 1024
B = 1024
T = 50
NC = T - 2 * CONTEXT
NSLOT = 8
TSPLIT = 24
CSPLIT = 20

_SC_INFO = plsc.get_sparse_core_info()
NCORE = _SC_INFO.num_cores
NSUB = _SC_INFO.num_subcores
NW = NCORE * NSUB
CHUNK = 64


def _make_gather(nslabs):
    nrows = nslabs * B
    rpw = nrows // NW
    nch = rpw // CHUNK

    def body(idx_hbm, table_hbm, out_hbm, idx_v, buf0, buf1, sem0, sem1):
        wid = lax.axis_index("s") * NCORE + lax.axis_index("c")
        pltpu.sync_copy(idx_hbm.at[wid], idx_v)
        bufs = (buf0, buf1)
        sems = (sem0, sem1)
        handles = [None] * nch
        handles[0] = pltpu.async_copy(table_hbm.at[idx_v.at[0]], buf0, sem0)
        for c in range(nch):
            if c + 1 < nch:
                handles[c + 1] = pltpu.async_copy(
                    table_hbm.at[idx_v.at[c + 1]], bufs[(c + 1) % 2], sems[(c + 1) % 2]
                )
            handles[c].wait()
            row0 = pl.multiple_of(wid * rpw + c * CHUNK, 8)
            pltpu.sync_copy(bufs[c % 2], out_hbm.at[pl.ds(row0, CHUNK)])

    return pl.kernel(
        body,
        mesh=plsc.VectorSubcoreMesh(core_axis_name="c", subcore_axis_name="s"),
        out_type=jax.ShapeDtypeStruct((nrows, EMB), jnp.float32),
        scratch_types=[
            pltpu.VMEM((nch, CHUNK), jnp.int32),
            pltpu.VMEM((CHUNK, EMB), jnp.float32),
            pltpu.VMEM((CHUNK, EMB), jnp.float32),
            pltpu.SemaphoreType.DMA,
            pltpu.SemaphoreType.DMA,
        ],
    )


_gather1 = _make_gather(TSPLIT)
_gather2 = _make_gather(T - TSPLIT)


def _slab_copy(e_hbm, ring, sems, t, base=0):
    return pltpu.make_async_copy(
        e_hbm.at[t - base], ring.at[t % NSLOT], sems.at[t % NSLOT]
    )


def _two_center_mlp(w1_ref, b1_ref, w2_ref, b2_ref, out_ref, ring, t0):
    def mlp(ctx):
        h = jax.lax.dot_general(
            w1_ref[...], ctx.astype(jnp.bfloat16), (((1,), (1,)), ((), ())),
            preferred_element_type=jnp.float32,
        )
        h = jnp.maximum(h + b1_ref[...], 0.0)
        out = jax.lax.dot_general(
            w2_ref[...], h.astype(jnp.bfloat16), (((1,), (0,)), ((), ())),
            preferred_element_type=jnp.float32,
        )
        return out + b2_ref[...]

    shared = ring[(t0 + 1) % NSLOT] + ring[(t0 + 4) % NSLOT]
    out_ref[0] = mlp(shared + ring[t0 % NSLOT] + ring[(t0 + 3) % NSLOT])
    out_ref[1] = mlp(shared + ring[(t0 + 2) % NSLOT] + ring[(t0 + 5) % NSLOT])


def _mlp_body1(e1_hbm, w1_ref, b1_ref, w2_ref, b2_ref, out_ref, ring, sems):
    j = pl.program_id(0)
    t0 = 2 * j

    @pl.when(j == 0)
    def _prime():
        for t in range(6):
            _slab_copy(e1_hbm, ring, sems, t).start()
        for t in range(6):
            _slab_copy(e1_hbm, ring, sems, t).wait()

    @pl.when(t0 + 7 < TSPLIT)
    def _prefetch():
        _slab_copy(e1_hbm, ring, sems, t0 + 6).start()
        _slab_copy(e1_hbm, ring, sems, t0 + 7).start()

    @pl.when(j > 0)
    def _await_slab():
        _slab_copy(e1_hbm, ring, sems, t0 + 4).wait()
        _slab_copy(e1_hbm, ring, sems, t0 + 5).wait()

    _two_center_mlp(w1_ref, b1_ref, w2_ref, b2_ref, out_ref, ring, t0)


def _mlp_body2(out_hbm, e1_hbm, e2_hbm, w1_ref, b1_ref, w2_ref, b2_ref,
               out_ref, ring, sems):
    del out_hbm
    j = pl.program_id(0)
    t0 = CSPLIT + 2 * j

    @pl.when(j == 0)
    def _prime():
        for t in range(CSPLIT, CSPLIT + 6):
            src, base = (e1_hbm, 0) if t < TSPLIT else (e2_hbm, TSPLIT)
            _slab_copy(src, ring, sems, t, base).start()
        for t in range(CSPLIT, CSPLIT + 6):
            src, base = (e1_hbm, 0) if t < TSPLIT else (e2_hbm, TSPLIT)
            _slab_copy(src, ring, sems, t, base).wait()

    @pl.when(t0 + 7 < T)
    def _prefetch():
        _slab_copy(e2_hbm, ring, sems, t0 + 6, TSPLIT).start()
        _slab_copy(e2_hbm, ring, sems, t0 + 7, TSPLIT).start()

    @pl.when(j > 0)
    def _await_slab():
        _slab_copy(e2_hbm, ring, sems, t0 + 4, TSPLIT).wait()
        _slab_copy(e2_hbm, ring, sems, t0 + 5, TSPLIT).wait()

    _two_center_mlp(w1_ref, b1_ref, w2_ref, b2_ref, out_ref, ring, t0)


@jax.jit
def kernel(batchCode, table, W1, b1, W2, b2):
    idx_t = batchCode.astype(jnp.int32).T
    idx1 = idx_t[:TSPLIT].reshape(NW, -1, CHUNK)
    idx2 = idx_t[TSPLIT:].reshape(NW, -1, CHUNK)
    e1 = _gather1(idx1, table).reshape(TSPLIT, B, EMB)
    e2 = _gather2(idx2, table).reshape(T - TSPLIT, B, EMB)

    w1t = W1.T.astype(jnp.bfloat16)
    b1c = b1.reshape(HID, 1)
    w2t = W2.T.astype(jnp.bfloat16)
    b2c = b2.reshape(UNIQUE_TOKENS, 1)
    wspecs = [
        pl.BlockSpec((HID, EMB), lambda i: (0, 0)),
        pl.BlockSpec((HID, 1), lambda i: (0, 0)),
        pl.BlockSpec((UNIQUE_TOKENS, HID), lambda i: (0, 0)),
        pl.BlockSpec((UNIQUE_TOKENS, 1), lambda i: (0, 0)),
    ]
    scratch = [
        pltpu.VMEM((NSLOT, B, EMB), jnp.float32),
        pltpu.SemaphoreType.DMA((NSLOT,)),
    ]
    out_shape = jax.ShapeDtypeStruct((NC, UNIQUE_TOKENS, B), jnp.float32)

    part1 = pl.pallas_call(
        _mlp_body1,
        grid=(CSPLIT // 2,),
        in_specs=[pl.BlockSpec(memory_space=pl.ANY)] + wspecs,
        out_specs=pl.BlockSpec((2, UNIQUE_TOKENS, B), lambda i: (i, 0, 0)),
        out_shape=out_shape,
        scratch_shapes=scratch,
    )(e1, w1t, b1c, w2t, b2c)

    out_t = pl.pallas_call(
        _mlp_body2,
        grid=((NC - CSPLIT) // 2,),
        in_specs=[pl.BlockSpec(memory_space=pl.ANY)] * 3 + wspecs,
        out_specs=pl.BlockSpec(
            (2, UNIQUE_TOKENS, B), lambda i: (i + CSPLIT // 2, 0, 0)
        ),
        out_shape=out_shape,
        input_output_aliases={0: 0},
        scratch_shapes=scratch,
    )(part1, e1, e2, w1t, b1c, w2t, b2c)

    return jnp.transpose(out_t, (2, 0, 1))
